# Initial kernel scaffold; baseline (speedup 1.0000x reference)
import jax
import jax.numpy as jnp
from jax.experimental import pallas as pl


def kernel(pos, centroids, centroids_index, index_voxels):
    raise NotImplementedError("write your pallas kernel here")


# speedup vs baseline: 8.6981x; 8.6981x over previous
import jax
import jax.numpy as jnp
import numpy as np
from jax import lax
from jax.experimental import pallas as pl
from jax.experimental.pallas import tpu as pltpu
from jax.experimental.pallas import tpu_sc as plsc

B = 8
N = 4096
S = 512
K = 32
L = 16
NC = 2
NS = 16
NW = NC * NS
QPW = (B * S) // NW
QBLK = S // QPW
CHUNKS = N // L
RSQ = np.float32(0.2 * 0.2)


def _ball_query_body(posx_hbm, cents_hbm, out_hbm,
                     x_v, y_v, z_v, pp_v, cid_v,
                     cxs_v, cys_v, czs_v, ccs_v, nbr_v, outb_v):
    cid = lax.axis_index("c")
    sid = lax.axis_index("s")
    wid = sid * NC + cid
    b = wid // QBLK
    qbase = (wid % QBLK) * QPW

    pltpu.sync_copy(posx_hbm.at[pl.ds((b * 3 + 0) * N, N)], x_v)
    pltpu.sync_copy(posx_hbm.at[pl.ds((b * 3 + 1) * N, N)], y_v)
    pltpu.sync_copy(posx_hbm.at[pl.ds((b * 3 + 2) * N, N)], z_v)
    pltpu.sync_copy(cents_hbm.at[pl.ds(b * S + qbase, QPW)], cid_v)

    def bf16r(v):
        bi = lax.bitcast_convert_type(v, jnp.int32)
        r = bi + jnp.int32(0x7FFF) + ((bi >> 16) & 1)
        return lax.bitcast_convert_type(r & jnp.int32(-65536), jnp.float32)

    def center_step(i, _):
        s = pl.ds(i * L, L)
        idxv = cid_v[s]
        cxv = plsc.load_gather(x_v, [idxv])
        cyv = plsc.load_gather(y_v, [idxv])
        czv = plsc.load_gather(z_v, [idxv])
        ccs_v[s] = (cxv * cxv + cyv * cyv) + czv * czv
        cxs_v[s] = bf16r(cxv)
        cys_v[s] = bf16r(cyv)
        czs_v[s] = bf16r(czv)
        return 0

    lax.fori_loop(0, QPW // L, center_step, 0)

    def pp_step(i, _):
        s = pl.ds(i * L, L)
        xv = x_v[s]
        yv = y_v[s]
        zv = z_v[s]
        pp_v[s] = (xv * xv + yv * yv) + zv * zv
        x_v[s] = bf16r(xv)
        y_v[s] = bf16r(yv)
        z_v[s] = bf16r(zv)
        return 0

    lax.fori_loop(0, CHUNKS, pp_step, 0)

    lanes = lax.broadcasted_iota(jnp.int32, (L,), 0)

    def one_query(q, _):
        cx = cxs_v[pl.ds(q, L)][0]
        cy = cys_v[pl.ds(q, L)][0]
        cz = czs_v[pl.ds(q, L)][0]
        cc = ccs_v[pl.ds(q, L)][0]

        def cond(carry):
            chunk, cnt = carry
            return jnp.logical_and(chunk < CHUNKS, cnt < K)

        def body(carry):
            chunk, cnt = carry
            base = chunk * L
            s = pl.ds(base, L)
            t = (cx * x_v[s] + cy * y_v[s]) + cz * z_v[s]
            d = (t * np.float32(-2.0) + cc) + pp_v[s]
            m = d <= RSQ
            plsc.store_compressed(nbr_v.at[pl.ds(cnt, L)], lanes + base, mask=m)
            cnt = cnt + jnp.sum(jnp.where(m, 1, 0).astype(jnp.int32))
            return chunk + 1, cnt

        _, cnt = lax.while_loop(cond, body, (jnp.int32(0), jnp.int32(0)))

        v0 = nbr_v[pl.ds(0, L)]
        v1 = nbr_v[pl.ds(L, L)]
        first = v0[0]
        outb_v[pl.ds(q * K, L)] = jnp.where(lanes < cnt, v0, first)
        outb_v[pl.ds(q * K + L, L)] = jnp.where(lanes + L < cnt, v1, first)
        return 0

    lax.fori_loop(0, QPW, one_query, 0)

    pltpu.sync_copy(outb_v, out_hbm.at[pl.ds((b * S + qbase) * K, QPW * K)])


@jax.jit
def _ball_query(posx, centroids):
    mesh = plsc.VectorSubcoreMesh(core_axis_name="c", subcore_axis_name="s")
    run = pl.kernel(
        _ball_query_body,
        out_type=jax.ShapeDtypeStruct((B * S * K,), jnp.int32),
        mesh=mesh,
        compiler_params=pltpu.CompilerParams(needs_layout_passes=False),
        scratch_types=[
            pltpu.VMEM((N,), jnp.float32),
            pltpu.VMEM((N,), jnp.float32),
            pltpu.VMEM((N,), jnp.float32),
            pltpu.VMEM((N,), jnp.float32),
            pltpu.VMEM((QPW,), jnp.int32),
            pltpu.VMEM((QPW + L,), jnp.float32),
            pltpu.VMEM((QPW + L,), jnp.float32),
            pltpu.VMEM((QPW + L,), jnp.float32),
            pltpu.VMEM((QPW + L,), jnp.float32),
            pltpu.VMEM((K + 2 * L,), jnp.int32),
            pltpu.VMEM((QPW * K,), jnp.int32),
        ],
    )
    return run(posx, centroids).reshape(B, S, K)


def kernel(pos, centroids, centroids_index, index_voxels):
    del centroids_index, index_voxels
    posx = jnp.transpose(pos, (0, 2, 1)).reshape(-1)
    return _ball_query(posx, centroids.reshape(-1))


# speedup vs baseline: 14.0138x; 1.6111x over previous
import jax
import jax.numpy as jnp
import numpy as np
from jax import lax
from jax.experimental import pallas as pl
from jax.experimental.pallas import tpu as pltpu
from jax.experimental.pallas import tpu_sc as plsc

B = 8
N = 4096
S = 512
K = 32
L = 16
NC = 2
NS = 16
NW = NC * NS
QPW = (B * S) // NW
QBLK = S // QPW
CHUNKS = N // L
UNROLL = 4
RSQ = np.float32(0.2 * 0.2)


def _ball_query_body(posx_hbm, cents_hbm, out_hbm,
                     x_v, y_v, z_v, pp_v, cid_v,
                     cxs_v, cys_v, czs_v, ccs_v, nbr_v, outb_v):
    cid = lax.axis_index("c")
    sid = lax.axis_index("s")
    wid = sid * NC + cid
    b = wid // QBLK
    qbase = (wid % QBLK) * QPW

    pltpu.sync_copy(posx_hbm.at[pl.ds((b * 3 + 0) * N, N)], x_v)
    pltpu.sync_copy(posx_hbm.at[pl.ds((b * 3 + 1) * N, N)], y_v)
    pltpu.sync_copy(posx_hbm.at[pl.ds((b * 3 + 2) * N, N)], z_v)
    pltpu.sync_copy(cents_hbm.at[pl.ds(b * S + qbase, QPW)], cid_v)

    def bf16r(v):
        bi = lax.bitcast_convert_type(v, jnp.int32)
        r = bi + jnp.int32(0x7FFF) + ((bi >> 16) & 1)
        return lax.bitcast_convert_type(r & jnp.int32(-65536), jnp.float32)

    def center_step(i, _):
        s = pl.ds(i * L, L)
        idxv = cid_v[s]
        cxv = plsc.load_gather(x_v, [idxv])
        cyv = plsc.load_gather(y_v, [idxv])
        czv = plsc.load_gather(z_v, [idxv])
        ccs_v[s] = (cxv * cxv + cyv * cyv) + czv * czv
        cxs_v[s] = bf16r(cxv)
        cys_v[s] = bf16r(cyv)
        czs_v[s] = bf16r(czv)
        return 0

    lax.fori_loop(0, QPW // L, center_step, 0)

    def pp_step(i, _):
        s = pl.ds(i * L, L)
        xv = x_v[s]
        yv = y_v[s]
        zv = z_v[s]
        pp_v[s] = (xv * xv + yv * yv) + zv * zv
        x_v[s] = bf16r(xv)
        y_v[s] = bf16r(yv)
        z_v[s] = bf16r(zv)
        return 0

    lax.fori_loop(0, CHUNKS, pp_step, 0)

    lanes = lax.broadcasted_iota(jnp.int32, (L,), 0)

    def one_query(q, _):
        cx = cxs_v[pl.ds(q, L)][0]
        cy = cys_v[pl.ds(q, L)][0]
        cz = czs_v[pl.ds(q, L)][0]
        cc = ccs_v[pl.ds(q, L)][0]

        def cond(carry):
            chunk, cnt = carry
            return jnp.logical_and(chunk < CHUNKS, cnt < K)

        def body(carry):
            chunk, cnt = carry
            base = chunk * L
            for u in range(UNROLL):
                s = pl.ds(base + u * L, L)
                t = (cx * x_v[s] + cy * y_v[s]) + cz * z_v[s]
                d = (t * np.float32(-2.0) + cc) + pp_v[s]
                m = d <= RSQ
                plsc.store_compressed(nbr_v.at[pl.ds(cnt, L)],
                                      lanes + (base + u * L), mask=m)
                cnt = cnt + plsc.all_reduce_population_count(m)[0]
            return chunk + UNROLL, cnt

        _, cnt = lax.while_loop(cond, body, (jnp.int32(0), jnp.int32(0)))

        v0 = nbr_v[pl.ds(0, L)]
        v1 = nbr_v[pl.ds(L, L)]
        first = v0[0]
        outb_v[pl.ds(q * K, L)] = jnp.where(lanes < cnt, v0, first)
        outb_v[pl.ds(q * K + L, L)] = jnp.where(lanes + L < cnt, v1, first)
        return 0

    lax.fori_loop(0, QPW, one_query, 0)

    pltpu.sync_copy(outb_v, out_hbm.at[pl.ds((b * S + qbase) * K, QPW * K)])


@jax.jit
def _ball_query(posx, centroids):
    mesh = plsc.VectorSubcoreMesh(core_axis_name="c", subcore_axis_name="s")
    run = pl.kernel(
        _ball_query_body,
        out_type=jax.ShapeDtypeStruct((B * S * K,), jnp.int32),
        mesh=mesh,
        compiler_params=pltpu.CompilerParams(needs_layout_passes=False),
        scratch_types=[
            pltpu.VMEM((N,), jnp.float32),
            pltpu.VMEM((N,), jnp.float32),
            pltpu.VMEM((N,), jnp.float32),
            pltpu.VMEM((N,), jnp.float32),
            pltpu.VMEM((QPW,), jnp.int32),
            pltpu.VMEM((QPW + L,), jnp.float32),
            pltpu.VMEM((QPW + L,), jnp.float32),
            pltpu.VMEM((QPW + L,), jnp.float32),
            pltpu.VMEM((QPW + L,), jnp.float32),
            pltpu.VMEM((K + (UNROLL + 1) * L,), jnp.int32),
            pltpu.VMEM((QPW * K,), jnp.int32),
        ],
    )
    return run(posx, centroids).reshape(B, S, K)


def kernel(pos, centroids, centroids_index, index_voxels):
    del centroids_index, index_voxels
    posx = jnp.transpose(pos, (0, 2, 1)).reshape(-1)
    return _ball_query(posx, centroids.reshape(-1))


# speedup vs baseline: 36.3196x; 2.5917x over previous
import jax
import jax.numpy as jnp
import numpy as np
from jax import lax
from jax.experimental import pallas as pl
from jax.experimental.pallas import tpu as pltpu
from jax.experimental.pallas import tpu_sc as plsc

B = 8
N = 4096
S = 512
K = 32
L = 16
NC = 2
NS = 16
NW = NC * NS
QPW = (B * S) // NW
QBLK = S // QPW
CHUNKS = N // L
UNROLL = 8
RSQ = np.float32(0.2 * 0.2)


def _ball_query_body(posx_hbm, cents_hbm, out_hbm,
                     x_v, y_v, z_v, pp_v, cid_v,
                     cxs_v, cys_v, czs_v, ccs_v, nbr_v, outb_v):
    cid = lax.axis_index("c")
    sid = lax.axis_index("s")
    wid = sid * NC + cid
    b = wid // QBLK
    qbase = (wid % QBLK) * QPW

    pltpu.sync_copy(posx_hbm.at[pl.ds((b * 3 + 0) * N, N)], x_v)
    pltpu.sync_copy(posx_hbm.at[pl.ds((b * 3 + 1) * N, N)], y_v)
    pltpu.sync_copy(posx_hbm.at[pl.ds((b * 3 + 2) * N, N)], z_v)
    pltpu.sync_copy(cents_hbm.at[pl.ds(b * S + qbase, QPW)], cid_v)

    def bf16r(v):
        bi = lax.bitcast_convert_type(v, jnp.int32)
        r = bi + jnp.int32(0x7FFF) + ((bi >> 16) & 1)
        return lax.bitcast_convert_type(r & jnp.int32(-65536), jnp.float32)

    def center_step(i, _):
        s = pl.ds(i * L, L)
        idxv = cid_v[s]
        cxv = plsc.load_gather(x_v, [idxv])
        cyv = plsc.load_gather(y_v, [idxv])
        czv = plsc.load_gather(z_v, [idxv])
        ccs_v[s] = (cxv * cxv + cyv * cyv) + czv * czv
        cxs_v[s] = bf16r(cxv)
        cys_v[s] = bf16r(cyv)
        czs_v[s] = bf16r(czv)
        return 0

    lax.fori_loop(0, QPW // L, center_step, 0)

    def pp_step(i, _):
        s = pl.ds(i * L, L)
        xv = x_v[s]
        yv = y_v[s]
        zv = z_v[s]
        pp_v[s] = (xv * xv + yv * yv) + zv * zv
        x_v[s] = bf16r(xv)
        y_v[s] = bf16r(yv)
        z_v[s] = bf16r(zv)
        return 0

    lax.fori_loop(0, CHUNKS, pp_step, 0)

    lanes = lax.broadcasted_iota(jnp.int32, (L,), 0)

    def one_query(q, _):
        cx = cxs_v[pl.ds(q, L)][0]
        cy = cys_v[pl.ds(q, L)][0]
        cz = czs_v[pl.ds(q, L)][0]
        cc = ccs_v[pl.ds(q, L)][0]

        def cond(carry):
            chunk, cnt = carry
            return jnp.logical_and(chunk < CHUNKS, cnt < K)

        def body(carry):
            chunk, cnt = carry
            base = chunk * L
            xs, ys, zs, ps = [], [], [], []
            for u in range(UNROLL):
                s = pl.ds(base + u * L, L)
                xs.append(x_v[s])
                ys.append(y_v[s])
                zs.append(z_v[s])
                ps.append(pp_v[s])
            ms = []
            for u in range(UNROLL):
                t = (cx * xs[u] + cy * ys[u]) + cz * zs[u]
                d = (t * np.float32(-2.0) + cc) + ps[u]
                ms.append(d <= RSQ)
            cs = [plsc.all_reduce_population_count(m)[0] for m in ms]
            offs = []
            for u in range(UNROLL):
                offs.append(cnt)
                cnt = cnt + cs[u]
            for u in range(UNROLL):
                plsc.store_compressed(nbr_v.at[pl.ds(offs[u], L)],
                                      lanes + (base + u * L), mask=ms[u])
            return chunk + UNROLL, cnt

        _, cnt = lax.while_loop(cond, body, (jnp.int32(0), jnp.int32(0)))

        v0 = nbr_v[pl.ds(0, L)]
        v1 = nbr_v[pl.ds(L, L)]
        first = v0[0]
        outb_v[pl.ds(q * K, L)] = jnp.where(lanes < cnt, v0, first)
        outb_v[pl.ds(q * K + L, L)] = jnp.where(lanes + L < cnt, v1, first)
        return 0

    lax.fori_loop(0, QPW, one_query, 0)

    pltpu.sync_copy(outb_v, out_hbm.at[pl.ds((b * S + qbase) * K, QPW * K)])


@jax.jit
def _ball_query(posx, centroids):
    mesh = plsc.VectorSubcoreMesh(core_axis_name="c", subcore_axis_name="s")
    run = pl.kernel(
        _ball_query_body,
        out_type=jax.ShapeDtypeStruct((B * S * K,), jnp.int32),
        mesh=mesh,
        compiler_params=pltpu.CompilerParams(needs_layout_passes=False),
        scratch_types=[
            pltpu.VMEM((N,), jnp.float32),
            pltpu.VMEM((N,), jnp.float32),
            pltpu.VMEM((N,), jnp.float32),
            pltpu.VMEM((N,), jnp.float32),
            pltpu.VMEM((QPW,), jnp.int32),
            pltpu.VMEM((QPW + L,), jnp.float32),
            pltpu.VMEM((QPW + L,), jnp.float32),
            pltpu.VMEM((QPW + L,), jnp.float32),
            pltpu.VMEM((QPW + L,), jnp.float32),
            pltpu.VMEM((K + (UNROLL + 1) * L,), jnp.int32),
            pltpu.VMEM((QPW * K,), jnp.int32),
        ],
    )
    return run(posx, centroids).reshape(B, S, K)


def kernel(pos, centroids, centroids_index, index_voxels):
    del centroids_index, index_voxels
    posx = jnp.transpose(pos, (0, 2, 1)).reshape(-1)
    return _ball_query(posx, centroids.reshape(-1))


# speedup vs baseline: 38.0977x; 1.0490x over previous
import jax
import jax.numpy as jnp
import numpy as np
from jax import lax
from jax.experimental import pallas as pl
from jax.experimental.pallas import tpu as pltpu
from jax.experimental.pallas import tpu_sc as plsc

B = 8
N = 4096
S = 512
K = 32
L = 16
NC = 2
NS = 16
NW = NC * NS
QPW = (B * S) // NW
QBLK = S // QPW
CHUNKS = N // L
UNROLL = 8
RSQ = np.float32(0.2 * 0.2)


def _ball_query_body(posx_hbm, posb_hbm, cents_hbm, out_hbm,
                     x_v, y_v, z_v, pp_v, xb_v, yb_v, zb_v, cid_v,
                     cxs_v, cys_v, czs_v, ccs_v, nbr_v, outb_v):
    cid = lax.axis_index("c")
    sid = lax.axis_index("s")
    wid = sid * NC + cid
    b = wid // QBLK
    qbase = (wid % QBLK) * QPW

    pltpu.sync_copy(posx_hbm.at[pl.ds((b * 3 + 0) * N, N)], x_v)
    pltpu.sync_copy(posx_hbm.at[pl.ds((b * 3 + 1) * N, N)], y_v)
    pltpu.sync_copy(posx_hbm.at[pl.ds((b * 3 + 2) * N, N)], z_v)
    pltpu.sync_copy(posb_hbm.at[pl.ds((b * 3 + 0) * N, N)], xb_v)
    pltpu.sync_copy(posb_hbm.at[pl.ds((b * 3 + 1) * N, N)], yb_v)
    pltpu.sync_copy(posb_hbm.at[pl.ds((b * 3 + 2) * N, N)], zb_v)
    pltpu.sync_copy(cents_hbm.at[pl.ds(b * S + qbase, QPW)], cid_v)

    def bf16r(v):
        bi = lax.bitcast_convert_type(v, jnp.int32)
        r = bi + jnp.int32(0x7FFF) + ((bi >> 16) & 1)
        return lax.bitcast_convert_type(r & jnp.int32(-65536), jnp.float32)

    def center_step(i, _):
        s = pl.ds(i * L, L)
        idxv = cid_v[s]
        cxv = plsc.load_gather(x_v, [idxv])
        cyv = plsc.load_gather(y_v, [idxv])
        czv = plsc.load_gather(z_v, [idxv])
        ccs_v[s] = (cxv * cxv + cyv * cyv) + czv * czv
        two = np.float32(2.0)
        cxs_v[s] = bf16r(cxv) * two
        cys_v[s] = bf16r(cyv) * two
        czs_v[s] = bf16r(czv) * two
        return 0

    lax.fori_loop(0, QPW // L, center_step, 0)

    def pp_step(i, _):
        s = pl.ds(i * L, L)
        xv = x_v[s]
        yv = y_v[s]
        zv = z_v[s]
        pp_v[s] = (xv * xv + yv * yv) + zv * zv
        return 0

    lax.fori_loop(0, CHUNKS, pp_step, 0)

    lanes = lax.broadcasted_iota(jnp.int32, (L,), 0)

    def one_query(q, _):
        cx = cxs_v[pl.ds(q, L)][0]
        cy = cys_v[pl.ds(q, L)][0]
        cz = czs_v[pl.ds(q, L)][0]
        cc = ccs_v[pl.ds(q, L)][0]

        def cond(carry):
            chunk, cnt = carry
            return jnp.logical_and(chunk < CHUNKS, cnt < K)

        def body(carry):
            chunk, cnt = carry
            base = chunk * L
            xs, ys, zs, ps = [], [], [], []
            for u in range(0, UNROLL, 2):
                s32 = pl.ds(base + u * L, 2 * L)
                xs.extend(plsc.unpack(xb_v[s32], format=plsc.PackFormat.INTERLEAVED))
                ys.extend(plsc.unpack(yb_v[s32], format=plsc.PackFormat.INTERLEAVED))
                zs.extend(plsc.unpack(zb_v[s32], format=plsc.PackFormat.INTERLEAVED))
            for u in range(UNROLL):
                ps.append(pp_v[pl.ds(base + u * L, L)])
            ms = []
            for u in range(UNROLL):
                t2 = (cx * xs[u] + cy * ys[u]) + cz * zs[u]
                d = (cc - t2) + ps[u]
                ms.append(d <= RSQ)
            cs = [plsc.all_reduce_population_count(m)[0] for m in ms]
            offs = []
            for u in range(UNROLL):
                offs.append(cnt)
                cnt = cnt + cs[u]
            for u in range(UNROLL):
                plsc.store_compressed(nbr_v.at[pl.ds(offs[u], L)],
                                      lanes + (base + u * L), mask=ms[u])
            return chunk + UNROLL, cnt

        _, cnt = lax.while_loop(cond, body, (jnp.int32(0), jnp.int32(0)))

        v0 = nbr_v[pl.ds(0, L)]
        v1 = nbr_v[pl.ds(L, L)]
        first = v0[0]
        outb_v[pl.ds(q * K, L)] = jnp.where(lanes < cnt, v0, first)
        outb_v[pl.ds(q * K + L, L)] = jnp.where(lanes + L < cnt, v1, first)
        return 0

    lax.fori_loop(0, QPW, one_query, 0)

    pltpu.sync_copy(outb_v, out_hbm.at[pl.ds((b * S + qbase) * K, QPW * K)])


@jax.jit
def _ball_query(posx, posb, centroids):
    mesh = plsc.VectorSubcoreMesh(core_axis_name="c", subcore_axis_name="s")
    run = pl.kernel(
        _ball_query_body,
        out_type=jax.ShapeDtypeStruct((B * S * K,), jnp.int32),
        mesh=mesh,
        compiler_params=pltpu.CompilerParams(needs_layout_passes=False),
        scratch_types=[
            pltpu.VMEM((N,), jnp.float32),
            pltpu.VMEM((N,), jnp.float32),
            pltpu.VMEM((N,), jnp.float32),
            pltpu.VMEM((N,), jnp.float32),
            pltpu.VMEM((N,), jnp.bfloat16),
            pltpu.VMEM((N,), jnp.bfloat16),
            pltpu.VMEM((N,), jnp.bfloat16),
            pltpu.VMEM((QPW,), jnp.int32),
            pltpu.VMEM((QPW + L,), jnp.float32),
            pltpu.VMEM((QPW + L,), jnp.float32),
            pltpu.VMEM((QPW + L,), jnp.float32),
            pltpu.VMEM((QPW + L,), jnp.float32),
            pltpu.VMEM((K + (UNROLL + 1) * L,), jnp.int32),
            pltpu.VMEM((QPW * K,), jnp.int32),
        ],
    )
    return run(posx, posb, centroids).reshape(B, S, K)


def kernel(pos, centroids, centroids_index, index_voxels):
    del centroids_index, index_voxels
    posx = jnp.transpose(pos, (0, 2, 1)).reshape(-1)
    posb = jnp.transpose(pos.astype(jnp.bfloat16), (0, 2, 1))
    posb = posb.reshape(B, 3, N // (2 * L), 2, L)
    posb = posb.transpose(0, 1, 2, 4, 3).reshape(-1)
    return _ball_query(posx, posb, centroids.reshape(-1))
